# baseline (speedup 1.0000x reference)
import dataclasses
import functools

import jax
import jax.numpy as jnp
from jax import lax
from jax.experimental import pallas as pl
from jax.experimental.pallas import tpu as pltpu
from jax.experimental.pallas import tpu_sc as plsc

_TOP = 4
_NP = 256
_NEG = -3.0e38


def _tc_body(n_real, bB, x0_ref, x1_ref, pc_ref, w0_ref, w1_ref, wp_ref,
             out_ref):
    inv_n = jnp.float32(1.0) / jnp.float32(n_real)

    p0 = jnp.sum(x0_ref[...], axis=1) * inv_n
    p1 = jnp.sum(x1_ref[...], axis=1) * inv_n
    pp = jnp.sum(pc_ref[...], axis=1, keepdims=True) * inv_n

    a0 = p0.astype(jnp.bfloat16)
    a1 = p1.astype(jnp.bfloat16)
    logits = jnp.dot(a0, w0_ref[...].astype(jnp.bfloat16),
                     preferred_element_type=jnp.float32)
    logits = logits + jnp.dot(a1, w1_ref[...].astype(jnp.bfloat16),
                              preferred_element_type=jnp.float32)
    wp = wp_ref[0:1, :].astype(jnp.bfloat16).astype(jnp.float32)
    logits = logits + pp.astype(jnp.bfloat16).astype(jnp.float32) * wp

    j = lax.broadcasted_iota(jnp.int32, logits.shape, 1)
    logits = jnp.where((j & (_NP - 1)) < n_real, logits, jnp.float32(_NEG))

    lanes = lax.broadcasted_iota(jnp.int32, (bB, 128), 1)
    buf = jnp.zeros((bB, 128), jnp.int32)
    for t in range(_TOP):
        pol = logits[:, t * _NP:(t + 1) * _NP]
        m = jnp.max(pol, axis=1, keepdims=True)
        pj = lax.broadcasted_iota(jnp.int32, pol.shape, 1)
        idx = jnp.min(jnp.where(pol == m, pj, _NP), axis=1, keepdims=True)
        buf = jnp.where(lanes == t, idx, buf)
    out_ref[...] = buf


def _tc_sample_indices(x0, x1, pctr_p, W0, W1, wp8):
    B, N, D = x0.shape
    T = _TOP
    bB = 64
    out = pl.pallas_call(
        functools.partial(_tc_body, N, bB),
        grid=(B // bB,),
        in_specs=[
            pl.BlockSpec((bB, N, D), lambda i: (i, 0, 0)),
            pl.BlockSpec((bB, N, D), lambda i: (i, 0, 0)),
            pl.BlockSpec((bB, _NP), lambda i: (i, 0)),
            pl.BlockSpec((D, T * _NP), lambda i: (0, 0)),
            pl.BlockSpec((D, T * _NP), lambda i: (0, 0)),
            pl.BlockSpec((8, T * _NP), lambda i: (0, 0)),
        ],
        out_specs=pl.BlockSpec((bB, 128), lambda i: (i, 0)),
        out_shape=jax.ShapeDtypeStruct((B, 128), jnp.int32),
    )(x0, x1, pctr_p, W0, W1, wp8)
    return out[:, :T]


def _sc_gather(pctr_p, idx2d, B):
    n_inst = 32
    rows_per = B // n_inst
    per = rows_per * _TOP
    npad = pctr_p.shape[1]
    mesh = plsc.VectorSubcoreMesh(core_axis_name="c", subcore_axis_name="s")
    cp = pltpu.CompilerParams()
    if "needs_layout_passes" in pltpu.CompilerParams.__dataclass_fields__:
        cp = dataclasses.replace(cp, needs_layout_passes=False)

    @pl.kernel(out_type=jax.ShapeDtypeStruct((n_inst, per), jnp.float32),
               mesh=mesh, compiler_params=cp,
               scratch_types=[
                   pltpu.VMEM((rows_per, npad), jnp.float32),
                   pltpu.VMEM((1, per), jnp.int32),
                   pltpu.VMEM((1, per), jnp.float32),
                   pltpu.SemaphoreType.DMA,
               ])
    def gather_kernel(pctr_hbm, idx_hbm, o_hbm, pc_vmem, idx_vmem, out_vmem,
                      sem):
        c = lax.axis_index("c")
        s = lax.axis_index("s")
        inst = c * 16 + s
        pltpu.async_copy(pctr_hbm.at[pl.ds(inst * rows_per, rows_per)],
                         pc_vmem, sem).wait()
        pltpu.async_copy(idx_hbm.at[pl.ds(inst, 1)], idx_vmem, sem).wait()
        for chunk in range(per // 16):
            p = chunk * 16 + lax.iota(jnp.int32, 16)
            row = lax.shift_right_logical(p, 2)
            col = idx_vmem[0, pl.ds(chunk * 16, 16)]
            vals = plsc.load_gather(pc_vmem, [row, col])
            out_vmem[0, pl.ds(chunk * 16, 16)] = vals
        pltpu.async_copy(out_vmem, o_hbm.at[pl.ds(inst, 1)], sem).wait()

    return gather_kernel(pctr_p, idx2d)


def kernel(x0, x1, pctr, W_gen):
    B, N, D = x0.shape
    T = _TOP

    Wr = W_gen.reshape(2 * D + 1, T, N + 1)[:, :, :N]
    Wf = jnp.pad(Wr, ((0, 0), (0, 0), (0, _NP - N))).reshape(2 * D + 1, T * _NP)
    W0 = Wf[:D]
    W1 = Wf[D:2 * D]
    wp8 = jnp.pad(Wf[2 * D][None, :], ((0, 7), (0, 0)))
    pctr_p = jnp.pad(pctr, ((0, 0), (0, _NP - N)))

    H = 2
    Bh = B // H
    parts = []
    for h in range(H):
        sl = slice(h * Bh, (h + 1) * Bh)
        idx = _tc_sample_indices(x0[sl], x1[sl], pctr_p[sl], W0, W1, wp8)
        idx2d = idx.reshape(32, (Bh // 32) * T)
        parts.append(_sc_gather(pctr_p[sl], idx2d, Bh).reshape(Bh, T))
    return jnp.concatenate(parts, axis=0)


# speedup vs baseline: 2.3138x; 2.3138x over previous
import dataclasses
import functools

import jax
import jax.numpy as jnp
from jax import lax
from jax.experimental import pallas as pl
from jax.experimental.pallas import tpu as pltpu
from jax.experimental.pallas import tpu_sc as plsc

_TOP = 4
_NP = 256
_NEG = -3.0e38


def _tc_body(n_real, x0_ref, x1_ref, pc_ref, w0_ref, w1_ref, wp_ref, out_ref):
    bB = x0_ref.shape[0]
    inv_n = jnp.float32(1.0) / jnp.float32(n_real)

    p0 = jnp.sum(x0_ref[...], axis=1) * inv_n
    p1 = jnp.sum(x1_ref[...], axis=1) * inv_n
    pp = jnp.sum(pc_ref[...], axis=1, keepdims=True) * inv_n

    a0 = p0.astype(jnp.bfloat16)
    a1 = p1.astype(jnp.bfloat16)
    logits = jnp.dot(a0, w0_ref[...].astype(jnp.bfloat16),
                     preferred_element_type=jnp.float32)
    logits = logits + jnp.dot(a1, w1_ref[...].astype(jnp.bfloat16),
                              preferred_element_type=jnp.float32)
    wp = wp_ref[0:1, :].astype(jnp.bfloat16).astype(jnp.float32)
    logits = logits + pp.astype(jnp.bfloat16).astype(jnp.float32) * wp

    j = lax.broadcasted_iota(jnp.int32, logits.shape, 1)
    logits = jnp.where((j & (_NP - 1)) < n_real, logits, jnp.float32(_NEG))

    lanes = lax.broadcasted_iota(jnp.int32, (bB, 128), 1)
    buf = jnp.zeros((bB, 128), jnp.int32)
    for t in range(_TOP):
        pol = logits[:, t * _NP:(t + 1) * _NP]
        m = jnp.max(pol, axis=1, keepdims=True)
        pj = lax.broadcasted_iota(jnp.int32, pol.shape, 1)
        idx = jnp.min(jnp.where(pol == m, pj, _NP), axis=1, keepdims=True)
        buf = jnp.where(lanes == t, idx, buf)
    out_ref[...] = buf


def _tc_sample_indices(x0, x1, pctr_p, W0, W1, wp8, blk0, nblk, bB):
    N, D = x0.shape[1], x0.shape[2]
    T = _TOP
    return pl.pallas_call(
        functools.partial(_tc_body, N),
        grid=(nblk,),
        in_specs=[
            pl.BlockSpec((bB, N, D), lambda i: (blk0 + i, 0, 0)),
            pl.BlockSpec((bB, N, D), lambda i: (blk0 + i, 0, 0)),
            pl.BlockSpec((bB, _NP), lambda i: (blk0 + i, 0)),
            pl.BlockSpec((D, T * _NP), lambda i: (0, 0)),
            pl.BlockSpec((D, T * _NP), lambda i: (0, 0)),
            pl.BlockSpec((8, T * _NP), lambda i: (0, 0)),
        ],
        out_specs=pl.BlockSpec((bB, 128), lambda i: (i, 0)),
        out_shape=jax.ShapeDtypeStruct((nblk * bB, 128), jnp.int32),
    )(x0, x1, pctr_p, W0, W1, wp8)


def _sc_gather(pctr_p, idxpad, row0, Bh):
    n_inst = 32
    rows_per = Bh // n_inst
    per = rows_per * _TOP
    npad = pctr_p.shape[1]
    mesh = plsc.VectorSubcoreMesh(core_axis_name="c", subcore_axis_name="s")
    cp = pltpu.CompilerParams()
    if "needs_layout_passes" in pltpu.CompilerParams.__dataclass_fields__:
        cp = dataclasses.replace(cp, needs_layout_passes=False)

    @pl.kernel(out_type=jax.ShapeDtypeStruct((n_inst, per), jnp.float32),
               mesh=mesh, compiler_params=cp,
               scratch_types=[
                   pltpu.VMEM((rows_per, npad), jnp.float32),
                   pltpu.VMEM((rows_per, 128), jnp.int32),
                   pltpu.VMEM((1, per), jnp.float32),
                   pltpu.SemaphoreType.DMA,
                   pltpu.SemaphoreType.DMA,
               ])
    def gather_kernel(pctr_hbm, idx_hbm, o_hbm, pc_vmem, idx_vmem, out_vmem,
                      sem0, sem1):
        c = lax.axis_index("c")
        s = lax.axis_index("s")
        inst = c * 16 + s
        cp0 = pltpu.async_copy(
            pctr_hbm.at[pl.ds(row0 + inst * rows_per, rows_per)],
            pc_vmem, sem0)
        cp1 = pltpu.async_copy(
            idx_hbm.at[pl.ds(inst * rows_per, rows_per)], idx_vmem, sem1)
        cp0.wait()
        cp1.wait()
        for chunk in range(per // 16):
            p = chunk * 16 + lax.iota(jnp.int32, 16)
            row = lax.shift_right_logical(p, 2)
            tlane = lax.bitwise_and(p, 3)
            col = plsc.load_gather(idx_vmem, [row, tlane])
            vals = plsc.load_gather(pc_vmem, [row, col])
            out_vmem[0, pl.ds(chunk * 16, 16)] = vals
        pltpu.async_copy(out_vmem, o_hbm.at[pl.ds(inst, 1)], sem0).wait()

    return gather_kernel(pctr_p, idxpad)


def kernel(x0, x1, pctr, W_gen):
    B, N, D = x0.shape
    T = _TOP

    Wr = W_gen.reshape(2 * D + 1, T, N + 1)[:, :, :N]
    Wf = jnp.pad(Wr, ((0, 0), (0, 0), (0, _NP - N))).reshape(2 * D + 1, T * _NP)
    W0 = Wf[:D]
    W1 = Wf[D:2 * D]
    wp8 = jnp.pad(Wf[2 * D][None, :], ((0, 7), (0, 0)))
    pctr_p = jnp.pad(pctr, ((0, 0), (0, _NP - N)))

    H = 2
    bB = 64
    Bh = B // H
    nblk = Bh // bB
    parts = []
    for h in range(H):
        idxpad = _tc_sample_indices(x0, x1, pctr_p, W0, W1, wp8,
                                    h * nblk, nblk, bB)
        parts.append(_sc_gather(pctr_p, idxpad, h * Bh, Bh).reshape(Bh, T))
    return jnp.concatenate(parts, axis=0)


# speedup vs baseline: 2.4050x; 1.0394x over previous
import dataclasses
import functools

import jax
import jax.numpy as jnp
from jax import lax
from jax.experimental import pallas as pl
from jax.experimental.pallas import tpu as pltpu
from jax.experimental.pallas import tpu_sc as plsc

_TOP = 4
_NP = 256
_NEG = -3.0e38


def _tc_body(n_real, x0_ref, x1_ref, pc_ref, w0_ref, w1_ref, wp_ref, out_ref):
    bB = x0_ref.shape[0]
    inv_n = jnp.float32(1.0) / jnp.float32(n_real)

    p0 = jnp.sum(x0_ref[...], axis=1) * inv_n
    p1 = jnp.sum(x1_ref[...], axis=1) * inv_n
    pp = jnp.sum(pc_ref[...], axis=1, keepdims=True) * inv_n

    a0 = p0.astype(jnp.bfloat16)
    a1 = p1.astype(jnp.bfloat16)
    logits = jnp.dot(a0, w0_ref[...].astype(jnp.bfloat16),
                     preferred_element_type=jnp.float32)
    logits = logits + jnp.dot(a1, w1_ref[...].astype(jnp.bfloat16),
                              preferred_element_type=jnp.float32)
    wp = wp_ref[0:1, :].astype(jnp.bfloat16).astype(jnp.float32)
    logits = logits + pp.astype(jnp.bfloat16).astype(jnp.float32) * wp

    j = lax.broadcasted_iota(jnp.int32, logits.shape, 1)
    logits = jnp.where((j & (_NP - 1)) < n_real, logits, jnp.float32(_NEG))

    lanes = lax.broadcasted_iota(jnp.int32, (bB, 128), 1)
    buf = jnp.zeros((bB, 128), jnp.int32)
    for t in range(_TOP):
        pol = logits[:, t * _NP:(t + 1) * _NP]
        m = jnp.max(pol, axis=1, keepdims=True)
        pj = lax.broadcasted_iota(jnp.int32, pol.shape, 1)
        idx = jnp.min(jnp.where(pol == m, pj, _NP), axis=1, keepdims=True)
        buf = jnp.where(lanes == t, idx, buf)
    out_ref[...] = buf


def _tc_sample_indices(x0, x1, pctr_p, W0, W1, wp8, blk0, nblk, bB):
    N, D = x0.shape[1], x0.shape[2]
    T = _TOP
    return pl.pallas_call(
        functools.partial(_tc_body, N),
        grid=(nblk,),
        in_specs=[
            pl.BlockSpec((bB, N, D), lambda i: (blk0 + i, 0, 0)),
            pl.BlockSpec((bB, N, D), lambda i: (blk0 + i, 0, 0)),
            pl.BlockSpec((bB, _NP), lambda i: (blk0 + i, 0)),
            pl.BlockSpec((D, T * _NP), lambda i: (0, 0)),
            pl.BlockSpec((D, T * _NP), lambda i: (0, 0)),
            pl.BlockSpec((8, T * _NP), lambda i: (0, 0)),
        ],
        out_specs=pl.BlockSpec((bB, 128), lambda i: (i, 0)),
        out_shape=jax.ShapeDtypeStruct((nblk * bB, 128), jnp.int32),
    )(x0, x1, pctr_p, W0, W1, wp8)


def _sc_gather(pctr_p, idxpad, row0, Bh):
    n_inst = 32
    rows_per = Bh // n_inst
    per = rows_per * _TOP
    npad = pctr_p.shape[1]
    mesh = plsc.VectorSubcoreMesh(core_axis_name="c", subcore_axis_name="s")
    cp = pltpu.CompilerParams()
    if "needs_layout_passes" in pltpu.CompilerParams.__dataclass_fields__:
        cp = dataclasses.replace(cp, needs_layout_passes=False)

    @pl.kernel(out_type=jax.ShapeDtypeStruct((n_inst, per), jnp.float32),
               mesh=mesh, compiler_params=cp,
               scratch_types=[
                   pltpu.VMEM((rows_per, npad), jnp.float32),
                   pltpu.VMEM((rows_per, 128), jnp.int32),
                   pltpu.VMEM((1, per), jnp.float32),
                   pltpu.SemaphoreType.DMA,
                   pltpu.SemaphoreType.DMA,
               ])
    def gather_kernel(pctr_hbm, idx_hbm, o_hbm, pc_vmem, idx_vmem, out_vmem,
                      sem0, sem1):
        c = lax.axis_index("c")
        s = lax.axis_index("s")
        inst = c * 16 + s
        cp0 = pltpu.async_copy(
            pctr_hbm.at[pl.ds(row0 + inst * rows_per, rows_per)],
            pc_vmem, sem0)
        cp1 = pltpu.async_copy(
            idx_hbm.at[pl.ds(inst * rows_per, rows_per)], idx_vmem, sem1)
        cp0.wait()
        cp1.wait()
        for chunk in range(per // 16):
            p = chunk * 16 + lax.iota(jnp.int32, 16)
            row = lax.shift_right_logical(p, 2)
            tlane = lax.bitwise_and(p, 3)
            col = plsc.load_gather(idx_vmem, [row, tlane])
            vals = plsc.load_gather(pc_vmem, [row, col])
            out_vmem[0, pl.ds(chunk * 16, 16)] = vals
        pltpu.async_copy(out_vmem, o_hbm.at[pl.ds(inst, 1)], sem0).wait()

    return gather_kernel(pctr_p, idxpad)


def kernel(x0, x1, pctr, W_gen):
    B, N, D = x0.shape
    T = _TOP

    Wr = W_gen.reshape(2 * D + 1, T, N + 1)[:, :, :N]
    Wf = jnp.pad(Wr, ((0, 0), (0, 0), (0, _NP - N))).reshape(2 * D + 1, T * _NP)
    W0 = Wf[:D]
    W1 = Wf[D:2 * D]
    wp8 = jnp.pad(Wf[2 * D][None, :], ((0, 7), (0, 0)))
    pctr_p = jnp.pad(pctr, ((0, 0), (0, _NP - N)))

    H = 1
    bB = 64
    Bh = B // H
    nblk = Bh // bB
    parts = []
    for h in range(H):
        idxpad = _tc_sample_indices(x0, x1, pctr_p, W0, W1, wp8,
                                    h * nblk, nblk, bB)
        parts.append(_sc_gather(pctr_p, idxpad, h * Bh, Bh).reshape(Bh, T))
    return jnp.concatenate(parts, axis=0)


# speedup vs baseline: 2.5275x; 1.0509x over previous
import dataclasses
import functools

import jax
import jax.numpy as jnp
from jax import lax
from jax.experimental import pallas as pl
from jax.experimental.pallas import tpu as pltpu
from jax.experimental.pallas import tpu_sc as plsc

_TOP = 4
_NP = 256
_NEG = -3.0e38


def _tc_body(n_real, x0_ref, x1_ref, pc_ref, w0_ref, w1_ref, wp_ref, out_ref):
    bB = x0_ref.shape[0]
    inv_n = jnp.float32(1.0) / jnp.float32(n_real)

    p0 = jnp.sum(x0_ref[...], axis=1) * inv_n
    p1 = jnp.sum(x1_ref[...], axis=1) * inv_n
    pp = jnp.sum(pc_ref[...], axis=1, keepdims=True) * inv_n

    a0 = p0.astype(jnp.bfloat16)
    a1 = p1.astype(jnp.bfloat16)
    logits = jnp.dot(a0, w0_ref[...].astype(jnp.bfloat16),
                     preferred_element_type=jnp.float32)
    logits = logits + jnp.dot(a1, w1_ref[...].astype(jnp.bfloat16),
                              preferred_element_type=jnp.float32)
    wp = wp_ref[0:1, :].astype(jnp.bfloat16).astype(jnp.float32)
    logits = logits + pp.astype(jnp.bfloat16).astype(jnp.float32) * wp

    j = lax.broadcasted_iota(jnp.int32, logits.shape, 1)
    logits = jnp.where((j & (_NP - 1)) < n_real, logits, jnp.float32(_NEG))

    lanes = lax.broadcasted_iota(jnp.int32, (bB, 128), 1)
    buf = jnp.zeros((bB, 128), jnp.int32)
    for t in range(_TOP):
        pol = logits[:, t * _NP:(t + 1) * _NP]
        m = jnp.max(pol, axis=1, keepdims=True)
        pj = lax.broadcasted_iota(jnp.int32, pol.shape, 1)
        idx = jnp.min(jnp.where(pol == m, pj, _NP), axis=1, keepdims=True)
        buf = jnp.where(lanes == t, idx, buf)
    out_ref[...] = buf


def _tc_sample_indices(x0, x1, pctr_p, W0, W1, wp8, blk0, nblk, bB):
    N, D = x0.shape[1], x0.shape[2]
    T = _TOP
    return pl.pallas_call(
        functools.partial(_tc_body, N),
        grid=(nblk,),
        in_specs=[
            pl.BlockSpec((bB, N, D), lambda i: (blk0 + i, 0, 0)),
            pl.BlockSpec((bB, N, D), lambda i: (blk0 + i, 0, 0)),
            pl.BlockSpec((bB, _NP), lambda i: (blk0 + i, 0)),
            pl.BlockSpec((D, T * _NP), lambda i: (0, 0)),
            pl.BlockSpec((D, T * _NP), lambda i: (0, 0)),
            pl.BlockSpec((8, T * _NP), lambda i: (0, 0)),
        ],
        out_specs=pl.BlockSpec((bB, 128), lambda i: (i, 0)),
        out_shape=jax.ShapeDtypeStruct((nblk * bB, 128), jnp.int32),
    )(x0, x1, pctr_p, W0, W1, wp8)


def _sc_gather(pctr_p, idxpad, row0, Bh):
    n_cores = 1
    n_inst = 16 * n_cores
    rows_per = Bh // n_inst
    per = rows_per * _TOP
    npad = pctr_p.shape[1]
    mesh = plsc.VectorSubcoreMesh(core_axis_name="c", subcore_axis_name="s",
                                  num_cores=n_cores)
    cp = pltpu.CompilerParams()
    if "needs_layout_passes" in pltpu.CompilerParams.__dataclass_fields__:
        cp = dataclasses.replace(cp, needs_layout_passes=False)

    @pl.kernel(out_type=jax.ShapeDtypeStruct((n_inst, per), jnp.float32),
               mesh=mesh, compiler_params=cp,
               scratch_types=[
                   pltpu.VMEM((rows_per, npad), jnp.float32),
                   pltpu.VMEM((rows_per, 128), jnp.int32),
                   pltpu.VMEM((1, per), jnp.float32),
                   pltpu.SemaphoreType.DMA,
                   pltpu.SemaphoreType.DMA,
               ])
    def gather_kernel(pctr_hbm, idx_hbm, o_hbm, pc_vmem, idx_vmem, out_vmem,
                      sem0, sem1):
        c = lax.axis_index("c")
        s = lax.axis_index("s")
        inst = c * 16 + s
        cp0 = pltpu.async_copy(
            pctr_hbm.at[pl.ds(row0 + inst * rows_per, rows_per)],
            pc_vmem, sem0)
        cp1 = pltpu.async_copy(
            idx_hbm.at[pl.ds(inst * rows_per, rows_per)], idx_vmem, sem1)
        cp0.wait()
        cp1.wait()
        for chunk in range(per // 16):
            p = chunk * 16 + lax.iota(jnp.int32, 16)
            row = lax.shift_right_logical(p, 2)
            tlane = lax.bitwise_and(p, 3)
            col = plsc.load_gather(idx_vmem, [row, tlane])
            vals = plsc.load_gather(pc_vmem, [row, col])
            out_vmem[0, pl.ds(chunk * 16, 16)] = vals
        pltpu.async_copy(out_vmem, o_hbm.at[pl.ds(inst, 1)], sem0).wait()

    return gather_kernel(pctr_p, idxpad)


def kernel(x0, x1, pctr, W_gen):
    B, N, D = x0.shape
    T = _TOP

    Wr = W_gen.reshape(2 * D + 1, T, N + 1)[:, :, :N]
    Wf = jnp.pad(Wr, ((0, 0), (0, 0), (0, _NP - N))).reshape(2 * D + 1, T * _NP)
    W0 = Wf[:D]
    W1 = Wf[D:2 * D]
    wp8 = jnp.pad(Wf[2 * D][None, :], ((0, 7), (0, 0)))
    pctr_p = jnp.pad(pctr, ((0, 0), (0, _NP - N)))

    H = 1
    bB = 64
    Bh = B // H
    nblk = Bh // bB
    parts = []
    for h in range(H):
        idxpad = _tc_sample_indices(x0, x1, pctr_p, W0, W1, wp8,
                                    h * nblk, nblk, bB)
        parts.append(_sc_gather(pctr_p, idxpad, h * Bh, Bh).reshape(Bh, T))
    return jnp.concatenate(parts, axis=0)
